# baseline (speedup 1.0000x reference)
import functools
import jax
import jax.numpy as jnp
from jax import lax
from jax.experimental import pallas as pl
from jax.experimental.pallas import tpu as pltpu
from jax.experimental.pallas import tpu_sc as plsc

B = 32
N_SNPS = 50000
N_GENES = 10000
N_NODES = 160000
CH = 16000
N_CHUNKS = N_NODES // CH
VECS = CH // 16


def _scale_body(snp_ref, f_ref, t_ref):
    fbar = jnp.mean(f_ref[...], axis=0, keepdims=True)
    t_ref[...] = snp_ref[...] * fbar


def _scaled_table(snp, filters):
    return pl.pallas_call(
        _scale_body,
        out_shape=jax.ShapeDtypeStruct((B, N_SNPS), jnp.float32),
    )(snp, filters)


def _perm(v, idx):
    dnums = lax.GatherDimensionNumbers(
        offset_dims=(), collapsed_slice_dims=(0,), start_index_map=(0,))
    return lax.gather(v, idx[:, None], dnums, (1,),
                      mode=lax.GatherScatterMode.PROMISE_IN_BOUNDS)


def _sc_segment_sum(t_tab, snp_ids, node_graph_ids):
    mesh = plsc.VectorSubcoreMesh(core_axis_name="c", subcore_axis_name="s")

    @functools.partial(
        pl.kernel,
        mesh=mesh,
        out_type=jax.ShapeDtypeStruct((B, N_GENES), jnp.float32),
        compiler_params=pltpu.CompilerParams(needs_layout_passes=False),
        scratch_types=[
            pltpu.VMEM((N_SNPS,), jnp.float32),
            pltpu.VMEM((N_GENES,), jnp.float32),
            pltpu.VMEM((CH,), jnp.int32),
            pltpu.VMEM((CH,), jnp.int32),
            pltpu.VMEM((CH + 32,), jnp.int32),
            pltpu.VMEM((CH + 32,), jnp.int32),
            pltpu.SemaphoreType.DMA,
            pltpu.SemaphoreType.DMA,
            pltpu.SemaphoreType.DMA,
            pltpu.SemaphoreType.DMA,
        ],
    )
    def sc_kernel(t_hbm, sid_hbm, gid_hbm, out_hbm, t_v, acc_v,
                  sid0, sid1, gid0, gid1, ss0, ss1, gs0, gs1):
        b = lax.axis_index("s") * 2 + lax.axis_index("c")
        sid_bufs = (sid0, sid1)
        gid_bufs = (gid0, gid1)
        ssem = (ss0, ss1)
        gsem = (gs0, gs1)

        def start(c, p):
            off = pl.multiple_of(c * CH, 8)
            pltpu.make_async_copy(
                sid_hbm.at[pl.ds(off, CH)], sid_bufs[p], ssem[p]).start()
            pltpu.make_async_copy(
                gid_hbm.at[pl.ds(off, CH)],
                gid_bufs[p].at[pl.ds(16, CH)], gsem[p]).start()

        def wait(p):
            pltpu.make_async_copy(
                sid_hbm.at[pl.ds(0, CH)], sid_bufs[p], ssem[p]).wait()
            pltpu.make_async_copy(
                gid_hbm.at[pl.ds(0, CH)],
                gid_bufs[p].at[pl.ds(16, CH)], gsem[p]).wait()

        start(0, 0)
        start(1, 1)
        pltpu.sync_copy(t_hbm.at[b], t_v)

        def zero(i, carry):
            acc_v[pl.ds(i * 16, 16)] = jnp.zeros((16,), jnp.float32)
            return carry
        lax.fori_loop(0, N_GENES // 16, zero, 0)

        iota = lax.iota(jnp.int32, 16)
        last = iota == 15

        def pair(c2, carry):
            for p in (0, 1):
                c = c2 * 2 + p
                wait(p)
                sid_v = sid_bufs[p]
                gid_v = gid_bufs[p]

                @plsc.parallel_loop(0, VECS, unroll=8)
                def vec(j):
                    base = 16 + j * 16
                    idx = sid_v[pl.ds(j * 16, 16)]
                    g = gid_v[pl.ds(base, 16)]
                    v = plsc.load_gather(t_v, [idx])
                    gp = _perm(g, jnp.maximum(iota - 1, 0))
                    gn = _perm(g, jnp.minimum(iota + 1, 15))
                    start = plsc.cummax(jnp.where(g != gp, iota, jnp.zeros((16,), jnp.int32)))
                    cs = plsc.cumsum(v)
                    sub = _perm(cs, jnp.maximum(start - 1, 0))
                    tot = cs - jnp.where(start > 0, sub, jnp.zeros((16,), jnp.float32))
                    end = (g != gn) | last
                    plsc.addupdate_scatter(acc_v, [g], tot, mask=end)

                @pl.when(c + 2 < N_CHUNKS)
                def _():
                    start(c + 2, p)
            return carry
        lax.fori_loop(0, N_CHUNKS // 2, pair, 0)

        pltpu.sync_copy(acc_v, out_hbm.at[b])

    return sc_kernel(t_tab, snp_ids, node_graph_ids)


def _mlp_body(sh_ref, w1_ref, b1_ref, g1_ref, bt1_ref,
              w2_ref, b2_ref, g2_ref, bt2_ref, w3_ref, b3_ref,
              out_ref, h1_scr):
    i = pl.program_id(0)
    z = jnp.dot(sh_ref[...], w1_ref[...], preferred_element_type=jnp.float32)
    z = z + b1_ref[...][None, :]
    m = jnp.mean(z, axis=0, keepdims=True)
    var = jnp.mean((z - m) * (z - m), axis=0, keepdims=True)
    h = g1_ref[...][None, :] * (z - m) / jnp.sqrt(var + 1e-5) + bt1_ref[...][None, :]
    h1_scr[:, pl.ds(i * 256, 256)] = jnp.maximum(h, 0.0)

    @pl.when(i == 3)
    def _():
        h1 = h1_scr[...]
        z2 = jnp.dot(h1, w2_ref[...], preferred_element_type=jnp.float32)
        z2 = z2 + b2_ref[...][None, :]
        m2 = jnp.mean(z2, axis=0, keepdims=True)
        v2 = jnp.mean((z2 - m2) * (z2 - m2), axis=0, keepdims=True)
        h2 = g2_ref[...][None, :] * (z2 - m2) / jnp.sqrt(v2 + 1e-5) + bt2_ref[...][None, :]
        h2 = jnp.maximum(h2, 0.0)
        out_ref[...] = jnp.dot(h2, w3_ref[...], preferred_element_type=jnp.float32) + b3_ref[...][None, :]


def _mlp(sample_h, W1, b1, gamma1, beta1, W2, b2, gamma2, beta2, W3, b3):
    return pl.pallas_call(
        _mlp_body,
        grid=(4,),
        in_specs=[
            pl.BlockSpec((B, N_GENES), lambda i: (0, 0)),
            pl.BlockSpec((N_GENES, 256), lambda i: (0, i)),
            pl.BlockSpec((256,), lambda i: (i,)),
            pl.BlockSpec((256,), lambda i: (i,)),
            pl.BlockSpec((256,), lambda i: (i,)),
            pl.BlockSpec((1024, 256), lambda i: (0, 0)),
            pl.BlockSpec((256,), lambda i: (0,)),
            pl.BlockSpec((256,), lambda i: (0,)),
            pl.BlockSpec((256,), lambda i: (0,)),
            pl.BlockSpec((256, 1), lambda i: (0, 0)),
            pl.BlockSpec((1,), lambda i: (0,)),
        ],
        out_specs=pl.BlockSpec((B, 1), lambda i: (0, 0)),
        out_shape=jax.ShapeDtypeStruct((B, 1), jnp.float32),
        scratch_shapes=[pltpu.VMEM((B, 1024), jnp.float32)],
    )(sample_h, W1, b1, gamma1, beta1, W2, b2, gamma2, beta2, W3, b3)


def kernel(snp, snp_ids, node_graph_ids, filters,
           W1, b1, gamma1, beta1, W2, b2, gamma2, beta2, W3, b3):
    t_tab = _scaled_table(snp, filters)
    sample_h = _sc_segment_sum(t_tab, snp_ids, node_graph_ids)
    preds = _mlp(sample_h, W1, b1, gamma1, beta1, W2, b2, gamma2, beta2, W3, b3)
    return (preds, filters)


# speedup vs baseline: 1.0011x; 1.0011x over previous
import functools
import jax
import jax.numpy as jnp
from jax import lax
from jax.experimental import pallas as pl
from jax.experimental.pallas import tpu as pltpu
from jax.experimental.pallas import tpu_sc as plsc

B = 32
N_SNPS = 50000
N_GENES = 10000
N_NODES = 160000
CH = 8000
N_CHUNKS = N_NODES // CH
VECS = CH // 16


def _scale_body(snp_ref, f_ref, t_ref):
    fbar = jnp.mean(f_ref[...], axis=0, keepdims=True)
    t_ref[...] = snp_ref[...] * fbar


def _scaled_table(snp, filters):
    return pl.pallas_call(
        _scale_body,
        out_shape=jax.ShapeDtypeStruct((B, N_SNPS), jnp.float32),
    )(snp, filters)


def _perm(v, idx):
    dnums = lax.GatherDimensionNumbers(
        offset_dims=(), collapsed_slice_dims=(0,), start_index_map=(0,))
    return lax.gather(v, idx[:, None], dnums, (1,),
                      mode=lax.GatherScatterMode.PROMISE_IN_BOUNDS)


def _sc_segment_sum(t_tab, snp_ids, node_graph_ids):
    mesh = plsc.VectorSubcoreMesh(core_axis_name="c", subcore_axis_name="s")

    @functools.partial(
        pl.kernel,
        mesh=mesh,
        out_type=jax.ShapeDtypeStruct((B, N_GENES), jnp.float32),
        compiler_params=pltpu.CompilerParams(needs_layout_passes=False),
        scratch_types=[
            pltpu.VMEM((N_SNPS,), jnp.float32),
            pltpu.VMEM((N_GENES,), jnp.float32),
            pltpu.VMEM((CH,), jnp.int32),
            pltpu.VMEM((CH,), jnp.int32),
            pltpu.VMEM((CH + 32,), jnp.int32),
            pltpu.VMEM((CH + 32,), jnp.int32),
            pltpu.SemaphoreType.DMA,
            pltpu.SemaphoreType.DMA,
            pltpu.SemaphoreType.DMA,
            pltpu.SemaphoreType.DMA,
        ],
    )
    def sc_kernel(t_hbm, sid_hbm, gid_hbm, out_hbm, t_v, acc_v,
                  sid0, sid1, gid0, gid1, ss0, ss1, gs0, gs1):
        b = lax.axis_index("s") * 2 + lax.axis_index("c")
        sid_bufs = (sid0, sid1)
        gid_bufs = (gid0, gid1)
        ssem = (ss0, ss1)
        gsem = (gs0, gs1)

        def start(c, p):
            off = pl.multiple_of(c * CH, 8)
            pltpu.make_async_copy(
                sid_hbm.at[pl.ds(off, CH)], sid_bufs[p], ssem[p]).start()
            pltpu.make_async_copy(
                gid_hbm.at[pl.ds(off, CH)],
                gid_bufs[p].at[pl.ds(16, CH)], gsem[p]).start()

        def wait(p):
            pltpu.make_async_copy(
                sid_hbm.at[pl.ds(0, CH)], sid_bufs[p], ssem[p]).wait()
            pltpu.make_async_copy(
                gid_hbm.at[pl.ds(0, CH)],
                gid_bufs[p].at[pl.ds(16, CH)], gsem[p]).wait()

        start(0, 0)
        start(1, 1)
        pltpu.sync_copy(t_hbm.at[b], t_v)

        def zero(i, carry):
            acc_v[pl.ds(i * 16, 16)] = jnp.zeros((16,), jnp.float32)
            return carry
        lax.fori_loop(0, N_GENES // 16, zero, 0)

        iota = lax.iota(jnp.int32, 16)
        last = iota == 15

        def pair(c2, carry):
            for p in (0, 1):
                c = c2 * 2 + p
                wait(p)
                sid_v = sid_bufs[p]
                gid_v = gid_bufs[p]

                @plsc.parallel_loop(0, VECS, unroll=8)
                def vec(j):
                    base = 16 + j * 16
                    idx = sid_v[pl.ds(j * 16, 16)]
                    g = gid_v[pl.ds(base, 16)]
                    v = plsc.load_gather(t_v, [idx])
                    gp = _perm(g, jnp.maximum(iota - 1, 0))
                    gn = _perm(g, jnp.minimum(iota + 1, 15))
                    start = plsc.cummax(jnp.where(g != gp, iota, jnp.zeros((16,), jnp.int32)))
                    cs = plsc.cumsum(v)
                    sub = _perm(cs, jnp.maximum(start - 1, 0))
                    tot = cs - jnp.where(start > 0, sub, jnp.zeros((16,), jnp.float32))
                    end = (g != gn) | last
                    plsc.addupdate_scatter(acc_v, [g], tot, mask=end)

                @pl.when(c + 2 < N_CHUNKS)
                def _():
                    start(c + 2, p)
            return carry
        lax.fori_loop(0, N_CHUNKS // 2, pair, 0)

        pltpu.sync_copy(acc_v, out_hbm.at[b])

    return sc_kernel(t_tab, snp_ids, node_graph_ids)


def _mlp_body(sh_ref, w1_ref, b1_ref, g1_ref, bt1_ref,
              w2_ref, b2_ref, g2_ref, bt2_ref, w3_ref, b3_ref,
              out_ref, h1_scr):
    i = pl.program_id(0)
    z = jnp.dot(sh_ref[...], w1_ref[...], preferred_element_type=jnp.float32)
    z = z + b1_ref[...][None, :]
    m = jnp.mean(z, axis=0, keepdims=True)
    var = jnp.mean((z - m) * (z - m), axis=0, keepdims=True)
    h = g1_ref[...][None, :] * (z - m) / jnp.sqrt(var + 1e-5) + bt1_ref[...][None, :]
    h1_scr[:, pl.ds(i * 256, 256)] = jnp.maximum(h, 0.0)

    @pl.when(i == 3)
    def _():
        h1 = h1_scr[...]
        z2 = jnp.dot(h1, w2_ref[...], preferred_element_type=jnp.float32)
        z2 = z2 + b2_ref[...][None, :]
        m2 = jnp.mean(z2, axis=0, keepdims=True)
        v2 = jnp.mean((z2 - m2) * (z2 - m2), axis=0, keepdims=True)
        h2 = g2_ref[...][None, :] * (z2 - m2) / jnp.sqrt(v2 + 1e-5) + bt2_ref[...][None, :]
        h2 = jnp.maximum(h2, 0.0)
        out_ref[...] = jnp.dot(h2, w3_ref[...], preferred_element_type=jnp.float32) + b3_ref[...][None, :]


def _mlp(sample_h, W1, b1, gamma1, beta1, W2, b2, gamma2, beta2, W3, b3):
    return pl.pallas_call(
        _mlp_body,
        grid=(4,),
        in_specs=[
            pl.BlockSpec((B, N_GENES), lambda i: (0, 0)),
            pl.BlockSpec((N_GENES, 256), lambda i: (0, i)),
            pl.BlockSpec((256,), lambda i: (i,)),
            pl.BlockSpec((256,), lambda i: (i,)),
            pl.BlockSpec((256,), lambda i: (i,)),
            pl.BlockSpec((1024, 256), lambda i: (0, 0)),
            pl.BlockSpec((256,), lambda i: (0,)),
            pl.BlockSpec((256,), lambda i: (0,)),
            pl.BlockSpec((256,), lambda i: (0,)),
            pl.BlockSpec((256, 1), lambda i: (0, 0)),
            pl.BlockSpec((1,), lambda i: (0,)),
        ],
        out_specs=pl.BlockSpec((B, 1), lambda i: (0, 0)),
        out_shape=jax.ShapeDtypeStruct((B, 1), jnp.float32),
        scratch_shapes=[pltpu.VMEM((B, 1024), jnp.float32)],
    )(sample_h, W1, b1, gamma1, beta1, W2, b2, gamma2, beta2, W3, b3)


def kernel(snp, snp_ids, node_graph_ids, filters,
           W1, b1, gamma1, beta1, W2, b2, gamma2, beta2, W3, b3):
    t_tab = _scaled_table(snp, filters)
    sample_h = _sc_segment_sum(t_tab, snp_ids, node_graph_ids)
    preds = _mlp(sample_h, W1, b1, gamma1, beta1, W2, b2, gamma2, beta2, W3, b3)
    return (preds, filters)
